# baseline (speedup 1.0000x reference)
import functools

import jax
import jax.numpy as jnp
from jax import lax
from jax.experimental import pallas as pl
from jax.experimental.pallas import tpu as pltpu
from jax.experimental.pallas import tpu_sc as plsc

B = 16384
EMB = 32
OUT = 128

NC = 2
NS = 16
NW = NC * NS

CH = 128
BPW = B // NW
CPW = BPW // CH


def _make_sc_gather():
    mesh = plsc.VectorSubcoreMesh(core_axis_name="c", subcore_axis_name="s")

    @functools.partial(
        pl.kernel,
        mesh=mesh,
        out_type=[jax.ShapeDtypeStruct((B, EMB), jnp.float32)] * 3,
        scratch_types=[
            pltpu.VMEM((CPW, CH), jnp.int32),
            pltpu.VMEM((CPW, CH), jnp.int32),
            pltpu.VMEM((CPW, CH), jnp.int32),
            pltpu.VMEM((BPW, EMB), jnp.float32),
            pltpu.VMEM((BPW, EMB), jnp.float32),
            pltpu.VMEM((BPW, EMB), jnp.float32),
            pltpu.SemaphoreType.DMA,
        ],
        compiler_params=pltpu.CompilerParams(use_tc_tiling_on_sc=False),
    )
    def gather_kernel(tid_hbm, did_hbm, bid_hbm, emb_t, emb_d, emb_b,
                      out_t, out_d, out_b,
                      it_v, id_v, ib_v, rt_v, rd_v, rb_v, sem):
        wid = lax.axis_index("s") * NC + lax.axis_index("c")
        cbase = wid * CPW
        pltpu.sync_copy(tid_hbm.at[pl.ds(cbase, CPW)], it_v)
        pltpu.sync_copy(did_hbm.at[pl.ds(cbase, CPW)], id_v)
        pltpu.sync_copy(bid_hbm.at[pl.ds(cbase, CPW)], ib_v)
        handles = []
        for j in range(CPW):
            dst = pl.ds(j * CH, CH)
            handles.append(pltpu.async_copy(emb_t.at[it_v.at[j]], rt_v.at[dst], sem))
            handles.append(pltpu.async_copy(emb_d.at[id_v.at[j]], rd_v.at[dst], sem))
            handles.append(pltpu.async_copy(emb_b.at[ib_v.at[j]], rb_v.at[dst], sem))
        for h in handles:
            h.wait()
        rbase = wid * BPW
        pltpu.sync_copy(rt_v, out_t.at[pl.ds(rbase, BPW)])
        pltpu.sync_copy(rd_v, out_d.at[pl.ds(rbase, BPW)])
        pltpu.sync_copy(rb_v, out_b.at[pl.ds(rbase, BPW)])

    return gather_kernel


_sc_gather = _make_sc_gather()

BM = 1024


def _mlp_body(et, ed, eb, w1, b1, w2, b2, o):
    h = jnp.dot(et[...], w1[0:EMB, :], preferred_element_type=jnp.float32)
    h = h + jnp.dot(ed[...], w1[EMB:2 * EMB, :], preferred_element_type=jnp.float32)
    h = h + jnp.dot(eb[...], w1[2 * EMB:3 * EMB, :], preferred_element_type=jnp.float32)
    h = h + b1[...]
    h = h * jax.nn.sigmoid(h)
    o[...] = jnp.dot(h, w2[...], preferred_element_type=jnp.float32) + b2[...]


def _mlp(e_t, e_d, e_b, W1, b1, W2, b2):
    grid = (B // BM,)
    eb_spec = pl.BlockSpec((BM, EMB), lambda i: (i, 0))
    full = lambda shape: pl.BlockSpec(shape, lambda i: (0,) * len(shape))
    return pl.pallas_call(
        _mlp_body,
        grid=grid,
        in_specs=[
            eb_spec, eb_spec, eb_spec,
            full((3 * EMB, OUT)),
            full((1, OUT)),
            full((OUT, OUT)),
            full((1, OUT)),
        ],
        out_specs=pl.BlockSpec((BM, OUT), lambda i: (i, 0)),
        out_shape=jax.ShapeDtypeStruct((B, OUT), jnp.float32),
    )(e_t, e_d, e_b, W1, b1, W2, b2)


@jax.jit
def kernel(tissue_id, disease_id, batch_id, emb_tissue, emb_disease, emb_batch,
           W1, b1, W2, b2):
    tid = tissue_id.astype(jnp.int32).reshape(B // CH, CH)
    did = disease_id.astype(jnp.int32).reshape(B // CH, CH)
    bid = batch_id.astype(jnp.int32).reshape(B // CH, CH)
    e_t, e_d, e_b = _sc_gather(tid, did, bid, emb_tissue, emb_disease, emb_batch)
    return _mlp(e_t, e_d, e_b, W1, b1.reshape(1, OUT), W2, b2.reshape(1, OUT))
